# baseline (speedup 1.0000x reference)
import functools

import jax
import jax.numpy as jnp
from jax import lax
from jax.experimental import pallas as pl
from jax.experimental.pallas import tpu as pltpu
from jax.experimental.pallas import tpu_sc as plsc

_B = 1024
_L = 20
_D = 64
_NC = 2
_NS = 16
_NW = _NC * _NS
_BPW = _B // _NW
_SEQ_PW = _BPW * _L
_CH = 128
_NCH = _SEQ_PW // _CH
_P = 2 * _D


_N = 100000
_BRT = 512
_NPACK = (_N + _BRT - 1) // _BRT


def _tpack_body(t_ref, out_ref):
    a = t_ref[...]
    r = lax.broadcasted_iota(jnp.int32, (_BRT // 2, _BRT), 1)
    k = lax.broadcasted_iota(jnp.int32, (_BRT // 2, _BRT), 0)
    qe = (r == 2 * k).astype(jnp.float32)
    qo = (r == 2 * k + 1).astype(jnp.float32)
    e = lax.dot_general(qe, a, (((1,), (1,)), ((), ())),
                        preferred_element_type=jnp.float32)
    o = lax.dot_general(qo, a, (((1,), (1,)), ((), ())),
                        preferred_element_type=jnp.float32)
    out_ref[...] = jnp.concatenate([e, o], axis=1)


_tpack_call = pl.pallas_call(
    _tpack_body,
    grid=(_NPACK,),
    in_specs=[pl.BlockSpec((_D, _BRT), lambda j: (0, j))],
    out_specs=pl.BlockSpec((_BRT // 2, 2 * _D), lambda j: (j, 0)),
    out_shape=jax.ShapeDtypeStruct((_N // 2, 2 * _D), jnp.float32),
)


def _sc_gather_body(table2, seq, ids,
                    seq_rows, tgt_rows,
                    seq_idx_v, pair_idx_v, ids_v, tid_v, rows_v, tgt_v, sem):
    wid = lax.axis_index("s") * _NC + lax.axis_index("c")
    jb = wid * _BPW
    sb = wid * _SEQ_PW
    pltpu.sync_copy(seq.at[pl.ds(sb, _SEQ_PW)], seq_idx_v)
    pltpu.sync_copy(ids.at[pl.ds(jb, _BPW)], ids_v)
    for c in range(_SEQ_PW // 16):
        pair_idx_v[pl.ds(c * 16, 16)] = (
            seq_idx_v[pl.ds(c * 16, 16)] >> 1)
    for c in range(_BPW // 16):
        tid_v[pl.ds(c * 16, 16)] = ids_v[pl.ds(c * 16, 16)] >> 1
    copies = []
    for k in range(_NCH):
        copies.append(pltpu.async_copy(
            table2.at[pair_idx_v.at[pl.ds(k * _CH, _CH)]],
            rows_v.at[pl.ds(k * _CH, _CH)], sem))
    copies.append(pltpu.async_copy(table2.at[tid_v], tgt_v, sem))
    for cp in copies:
        cp.wait()
    pltpu.sync_copy(rows_v, seq_rows.at[pl.ds(sb, _SEQ_PW)])
    pltpu.sync_copy(tgt_v, tgt_rows.at[pl.ds(jb, _BPW)])


@functools.cache
def _sc_gather():
    return pl.kernel(
        _sc_gather_body,
        out_type=(jax.ShapeDtypeStruct((_B * _L, _P), jnp.float32),
                  jax.ShapeDtypeStruct((_B, _P), jnp.float32)),
        mesh=plsc.VectorSubcoreMesh(core_axis_name="c", subcore_axis_name="s"),
        scratch_types=[
            pltpu.VMEM((_SEQ_PW,), jnp.int32),
            pltpu.VMEM((_SEQ_PW,), jnp.int32),
            pltpu.VMEM((_BPW,), jnp.int32),
            pltpu.VMEM((_BPW,), jnp.int32),
            pltpu.VMEM((_SEQ_PW, _P), jnp.float32),
            pltpu.VMEM((_BPW, _P), jnp.float32),
            pltpu.SemaphoreType.DMA,
        ],
    )


_BJ = 128


def _dot_body(seq_ref, tgt_ref, sidx_ref, tidx_ref, out_ref):
    pr = seq_ref[...].reshape(_BJ, _L, _P)
    tp = tgt_ref[...].reshape(_BJ, 1, _P)
    a_i = lax.broadcasted_iota(jnp.int32, (_P, _P), 0)
    b_i = lax.broadcasted_iota(jnp.int32, (_P, _P), 1)
    r128 = (b_i == ((a_i + _D) % _P)).astype(jnp.float32)
    tp_roll = lax.dot_general(
        tgt_ref[...], r128, (((1,), (0,)), ((), ())),
        preferred_element_type=jnp.float32).reshape(_BJ, 1, _P)
    lane = lax.broadcasted_iota(jnp.int32, (1, 1, _P), 2)
    m0 = (lane < _D).astype(jnp.float32)
    a = pr * tp
    b = pr * tp_roll
    sa0 = jnp.sum(a * m0, axis=2)
    sa = jnp.sum(a, axis=2)
    sb0 = jnp.sum(b * m0, axis=2)
    sb = jnp.sum(b, axis=2)
    s00, s11 = sa0, sa - sa0
    s01, s10 = sb0, sb - sb0
    ps = sidx_ref[...] & 1
    pt = tidx_ref[...] & 1
    s2 = jnp.where(ps == pt,
                   jnp.where(ps == 0, s00, s11),
                   jnp.where(ps == 0, s01, s10))
    r = lax.broadcasted_iota(jnp.int32, (_L, _L), 0)
    c = lax.broadcasted_iota(jnp.int32, (_L, _L), 1)
    tri = (c <= r).astype(jnp.float32)
    out_ref[...] = lax.dot_general(
        tri, s2, (((1,), (1,)), ((), ())), preferred_element_type=jnp.float32)


_dot_call = pl.pallas_call(
    _dot_body,
    grid=(_B // _BJ,),
    in_specs=[
        pl.BlockSpec((_BJ * _L, _P), lambda j: (j, 0)),
        pl.BlockSpec((_BJ, _P), lambda j: (j, 0)),
        pl.BlockSpec((_BJ, _L), lambda j: (j, 0)),
        pl.BlockSpec((_BJ, 1), lambda j: (j, 0)),
    ],
    out_specs=pl.BlockSpec((_L, _BJ), lambda j: (0, j)),
    out_shape=jax.ShapeDtypeStruct((_L, _B), jnp.float32),
)


_BI = 64


def _bcast_body(dotT_ref, bias_ref, out_ref):
    d = dotT_ref[...]
    b = bias_ref[...]
    for l in range(_L):
        out_ref[l] = d[l:l + 1, :] + b


_bcast_call = pl.pallas_call(
    _bcast_body,
    grid=(_B // _BI,),
    in_specs=[
        pl.BlockSpec((_L, _B), lambda i: (0, 0)),
        pl.BlockSpec((_BI, 1), lambda i: (i, 0)),
    ],
    out_specs=pl.BlockSpec((_L, _BI, _B), lambda i: (0, i, 0)),
    out_shape=jax.ShapeDtypeStruct((_L, _B, _B), jnp.float32),
)


def kernel(item_sequences, item_ids, item_embeddings_weight, item_biases_weight):
    seq = item_sequences.reshape(-1)
    ids = item_ids.reshape(-1)
    table2 = _tpack_call(item_embeddings_weight.T)
    seq_rows, tgt_rows = _sc_gather()(table2, seq, ids)
    dotT = _dot_call(seq_rows, tgt_rows, item_sequences, item_ids)
    bias_g = item_biases_weight[ids]
    out_phys = _bcast_call(dotT, bias_g)
    return jnp.transpose(out_phys, (1, 2, 0))


# speedup vs baseline: 1.1056x; 1.1056x over previous
import functools

import jax
import jax.numpy as jnp
from jax import lax
from jax.experimental import pallas as pl
from jax.experimental.pallas import tpu as pltpu
from jax.experimental.pallas import tpu_sc as plsc

_B = 1024
_L = 20
_D = 64
_NC = 2
_NS = 16
_NW = _NC * _NS
_BPW = _B // _NW
_SEQ_PW = _BPW * _L
_CH = 128
_NCH = _SEQ_PW // _CH
_P = 2 * _D


_N = 100000
_BRT = 512
_NPACK = (_N + _BRT - 1) // _BRT


def _tpack_body(t_ref, out_ref):
    t2 = lax.transpose(t_ref[...], (1, 0))
    out_ref[...] = jnp.concatenate(
        [t2, jnp.zeros((_BRT, _D), jnp.float32)], axis=1)


_tpack_call = pl.pallas_call(
    _tpack_body,
    grid=(_NPACK,),
    in_specs=[pl.BlockSpec((_D, _BRT), lambda j: (0, j))],
    out_specs=pl.BlockSpec((_BRT, 2 * _D), lambda j: (j, 0)),
    out_shape=jax.ShapeDtypeStruct((_N, 2 * _D), jnp.float32),
)


def _sc_gather_body(table2, seq, ids,
                    seq_rows, tgt_rows,
                    seq_idx_v, ids_v, rows_v, tgt_v, sem):
    wid = lax.axis_index("s") * _NC + lax.axis_index("c")
    jb = wid * _BPW
    sb = wid * _SEQ_PW
    pltpu.sync_copy(seq.at[pl.ds(sb, _SEQ_PW)], seq_idx_v)
    pltpu.sync_copy(ids.at[pl.ds(jb, _BPW)], ids_v)
    copies = []
    for k in range(_NCH):
        copies.append(pltpu.async_copy(
            table2.at[seq_idx_v.at[pl.ds(k * _CH, _CH)]],
            rows_v.at[pl.ds(k * _CH, _CH)], sem))
    copies.append(pltpu.async_copy(table2.at[ids_v], tgt_v, sem))
    for cp in copies:
        cp.wait()
    pltpu.sync_copy(rows_v, seq_rows.at[pl.ds(sb, _SEQ_PW)])
    pltpu.sync_copy(tgt_v, tgt_rows.at[pl.ds(jb, _BPW)])


@functools.cache
def _sc_gather():
    return pl.kernel(
        _sc_gather_body,
        out_type=(jax.ShapeDtypeStruct((_B * _L, _P), jnp.float32),
                  jax.ShapeDtypeStruct((_B, _P), jnp.float32)),
        mesh=plsc.VectorSubcoreMesh(core_axis_name="c", subcore_axis_name="s"),
        scratch_types=[
            pltpu.VMEM((_SEQ_PW,), jnp.int32),
            pltpu.VMEM((_BPW,), jnp.int32),
            pltpu.VMEM((_SEQ_PW, _P), jnp.float32),
            pltpu.VMEM((_BPW, _P), jnp.float32),
            pltpu.SemaphoreType.DMA,
        ],
    )


_BJ = 128


def _dot_body(seq_ref, tgt_ref, out_ref):
    pr = seq_ref[...].reshape(_BJ, _L, _P)
    tp = tgt_ref[...].reshape(_BJ, 1, _P)
    s2 = jnp.sum(pr * tp, axis=2)
    r = lax.broadcasted_iota(jnp.int32, (_L, _L), 0)
    c = lax.broadcasted_iota(jnp.int32, (_L, _L), 1)
    tri = (c <= r).astype(jnp.float32)
    out_ref[...] = lax.dot_general(
        tri, s2, (((1,), (1,)), ((), ())), preferred_element_type=jnp.float32)


_dot_call = pl.pallas_call(
    _dot_body,
    grid=(_B // _BJ,),
    in_specs=[
        pl.BlockSpec((_BJ * _L, _P), lambda j: (j, 0)),
        pl.BlockSpec((_BJ, _P), lambda j: (j, 0)),
    ],
    out_specs=pl.BlockSpec((_L, _BJ), lambda j: (0, j)),
    out_shape=jax.ShapeDtypeStruct((_L, _B), jnp.float32),
)


_BI = 64


def _bcast_body(dotT_ref, bias_ref, out_ref):
    d = dotT_ref[...]
    b = bias_ref[...]
    for l in range(_L):
        out_ref[l] = d[l:l + 1, :] + b


_bcast_call = pl.pallas_call(
    _bcast_body,
    grid=(_B // _BI,),
    in_specs=[
        pl.BlockSpec((_L, _B), lambda i: (0, 0)),
        pl.BlockSpec((_BI, 1), lambda i: (i, 0)),
    ],
    out_specs=pl.BlockSpec((_L, _BI, _B), lambda i: (0, i, 0)),
    out_shape=jax.ShapeDtypeStruct((_L, _B, _B), jnp.float32),
)


def kernel(item_sequences, item_ids, item_embeddings_weight, item_biases_weight):
    seq = item_sequences.reshape(-1)
    ids = item_ids.reshape(-1)
    table2 = _tpack_call(item_embeddings_weight.T)
    seq_rows, tgt_rows = _sc_gather()(table2, seq, ids)
    dotT = _dot_call(seq_rows, tgt_rows)
    bias_g = item_biases_weight[ids]
    out_phys = _bcast_call(dotT, bias_g)
    return jnp.transpose(out_phys, (1, 2, 0))


# speedup vs baseline: 1.5872x; 1.4356x over previous
import functools

import jax
import jax.numpy as jnp
from jax import lax
from jax.experimental import pallas as pl
from jax.experimental.pallas import tpu as pltpu
from jax.experimental.pallas import tpu_sc as plsc

_B = 1024
_L = 20
_D = 64
_NC = 2
_NS = 16
_NW = _NC * _NS
_BPW = _B // _NW
_SEQ_PW = _BPW * _L
_CH = 128
_NCH = _SEQ_PW // _CH


_CHUNK = 32


_HB = _B // 2
_BPW_H = _HB // _NW
_SEQ_PW_H = _BPW_H * _L


def _sc_gather_body(h, table, seq, ids,
                    seq_rows, tgt_rows,
                    seq_idx_v, ids_v, rows_v, tgt_v, sem):
    wid = lax.axis_index("s") * _NC + lax.axis_index("c")
    jb = wid * _BPW_H
    sb = jb * _L
    jb_g = h * _HB + jb
    sb_g = jb_g * _L
    pltpu.sync_copy(seq.at[pl.ds(sb_g, _SEQ_PW_H)], seq_idx_v)
    pltpu.sync_copy(ids.at[pl.ds(jb_g, _BPW_H)], ids_v)
    pending = []
    for c in range(_SEQ_PW_H // 16):
        vec = seq_idx_v[pl.ds(c * 16, 16)]
        copies = []
        for u in range(16):
            i = c * 16 + u
            copies.append(pltpu.async_copy(
                table.at[pl.ds(vec[u], 1)], rows_v.at[pl.ds(i, 1)], sem))
        for cp in pending:
            cp.wait()
        pending = copies
    tcopies = []
    for c in range(_BPW_H // 16):
        vec = ids_v[pl.ds(c * 16, 16)]
        for u in range(16):
            i = c * 16 + u
            tcopies.append(pltpu.async_copy(
                table.at[pl.ds(vec[u], 1)], tgt_v.at[pl.ds(i, 1)], sem))
    for cp in pending:
        cp.wait()
    for cp in tcopies:
        cp.wait()
    pltpu.sync_copy(rows_v, seq_rows.at[pl.ds(sb, _SEQ_PW_H)])
    pltpu.sync_copy(tgt_v, tgt_rows.at[pl.ds(jb, _BPW_H)])


@functools.cache
def _sc_gather(h):
    return pl.kernel(
        functools.partial(_sc_gather_body, h),
        out_type=(jax.ShapeDtypeStruct((_HB * _L, _D), jnp.float32),
                  jax.ShapeDtypeStruct((_HB, _D), jnp.float32)),
        mesh=plsc.VectorSubcoreMesh(core_axis_name="c", subcore_axis_name="s"),
        scratch_types=[
            pltpu.VMEM((_SEQ_PW_H,), jnp.int32),
            pltpu.VMEM((_BPW_H,), jnp.int32),
            pltpu.VMEM((_SEQ_PW_H, _D), jnp.float32),
            pltpu.VMEM((_BPW_H, _D), jnp.float32),
            pltpu.SemaphoreType.DMA,
        ],
        name=f"sc_gather_h{h}",
    )


_BJ = 128


def _dot_body(seq_ref, tgt_ref, out_ref):
    s = seq_ref[...].reshape(_BJ, _L, _D)
    t = tgt_ref[...].reshape(_BJ, 1, _D)
    s2 = jnp.sum(s * t, axis=2)
    r = lax.broadcasted_iota(jnp.int32, (_L, _L), 0)
    c = lax.broadcasted_iota(jnp.int32, (_L, _L), 1)
    tri = (c <= r).astype(jnp.float32)
    out_ref[...] = lax.dot_general(
        tri, s2, (((1,), (1,)), ((), ())), preferred_element_type=jnp.float32)


_dot_call = pl.pallas_call(
    _dot_body,
    grid=(_HB // _BJ,),
    in_specs=[
        pl.BlockSpec((_BJ * _L, _D), lambda j: (j, 0)),
        pl.BlockSpec((_BJ, _D), lambda j: (j, 0)),
    ],
    out_specs=pl.BlockSpec((_L, _BJ), lambda j: (0, j)),
    out_shape=jax.ShapeDtypeStruct((_L, _HB), jnp.float32),
)


_BI = 64


def _bcast_body(dotT_ref, bias_ref, out_ref):
    d = dotT_ref[...]
    b = bias_ref[...]
    for l in range(_L):
        out_ref[l] = d[l:l + 1, :] + b


def _bcast_body2(dotT_ref, bias_ref, carry_ref, out_ref):
    del carry_ref
    _bcast_body(dotT_ref, bias_ref, out_ref)


_bcast_call1 = pl.pallas_call(
    _bcast_body,
    grid=(_B // _BI,),
    in_specs=[
        pl.BlockSpec((_L, _HB), lambda i: (0, 0)),
        pl.BlockSpec((_BI, 1), lambda i: (i, 0)),
    ],
    out_specs=pl.BlockSpec((_L, _BI, _HB), lambda i: (0, i, 0)),
    out_shape=jax.ShapeDtypeStruct((_L, _B, _B), jnp.float32),
)

_bcast_call2 = pl.pallas_call(
    _bcast_body2,
    grid=(_B // _BI,),
    in_specs=[
        pl.BlockSpec((_L, _HB), lambda i: (0, 0)),
        pl.BlockSpec((_BI, 1), lambda i: (i, 0)),
        pl.BlockSpec(memory_space=pltpu.MemorySpace.HBM),
    ],
    out_specs=pl.BlockSpec((_L, _BI, _HB), lambda i: (0, i, 1)),
    out_shape=jax.ShapeDtypeStruct((_L, _B, _B), jnp.float32),
    input_output_aliases={2: 0},
)


def kernel(item_sequences, item_ids, item_embeddings_weight, item_biases_weight):
    seq = item_sequences.reshape(-1)
    ids = item_ids.reshape(-1)
    sr1, tr1 = _sc_gather(0)(item_embeddings_weight, seq, ids)
    sr2, tr2 = _sc_gather(1)(item_embeddings_weight, seq, ids)
    d1 = _dot_call(sr1, tr1)
    d2 = _dot_call(sr2, tr2)
    bias_g = item_biases_weight[ids]
    out1 = _bcast_call1(d1, bias_g)
    out_phys = _bcast_call2(d2, bias_g, out1)
    return jnp.transpose(out_phys, (1, 2, 0))


# speedup vs baseline: 1.8382x; 1.1581x over previous
import functools

import jax
import jax.numpy as jnp
from jax import lax
from jax.experimental import pallas as pl
from jax.experimental.pallas import tpu as pltpu
from jax.experimental.pallas import tpu_sc as plsc

_B = 1024
_L = 20
_D = 64
_NC = 2
_NS = 16
_NW = _NC * _NS
_BPW = _B // _NW
_SEQ_PW = _BPW * _L
_CH = 128
_NCH = _SEQ_PW // _CH


_CHUNK = 32


def _sc_gather_body(table, seq, ids,
                    seq_rows, tgt_rows,
                    seq_idx_v, ids_v, rows_v, tgt_v, sem):
    wid = lax.axis_index("s") * _NC + lax.axis_index("c")
    jb = wid * _BPW
    sb = wid * _SEQ_PW
    pltpu.sync_copy(seq.at[pl.ds(sb, _SEQ_PW)], seq_idx_v)
    pltpu.sync_copy(ids.at[pl.ds(jb, _BPW)], ids_v)
    pending = []
    for c in range(_SEQ_PW // 16):
        vec = seq_idx_v[pl.ds(c * 16, 16)]
        copies = []
        for u in range(16):
            i = c * 16 + u
            copies.append(pltpu.async_copy(
                table.at[pl.ds(vec[u], 1)], rows_v.at[pl.ds(i, 1)], sem))
        pending.append(copies)
        if len(pending) > 3:
            for cp in pending.pop(0):
                cp.wait()
    pending = [cp for chunk in pending for cp in chunk]
    tcopies = []
    for c in range(_BPW // 16):
        vec = ids_v[pl.ds(c * 16, 16)]
        for u in range(16):
            i = c * 16 + u
            tcopies.append(pltpu.async_copy(
                table.at[pl.ds(vec[u], 1)], tgt_v.at[pl.ds(i, 1)], sem))
    for cp in pending:
        cp.wait()
    for cp in tcopies:
        cp.wait()
    pltpu.sync_copy(rows_v, seq_rows.at[pl.ds(sb, _SEQ_PW)])
    pltpu.sync_copy(tgt_v, tgt_rows.at[pl.ds(jb, _BPW)])


@functools.cache
def _sc_gather():
    return pl.kernel(
        _sc_gather_body,
        out_type=(jax.ShapeDtypeStruct((_B * _L, _D), jnp.float32),
                  jax.ShapeDtypeStruct((_B, _D), jnp.float32)),
        mesh=plsc.VectorSubcoreMesh(core_axis_name="c", subcore_axis_name="s"),
        scratch_types=[
            pltpu.VMEM((_SEQ_PW,), jnp.int32),
            pltpu.VMEM((_BPW,), jnp.int32),
            pltpu.VMEM((_SEQ_PW, _D), jnp.float32),
            pltpu.VMEM((_BPW, _D), jnp.float32),
            pltpu.SemaphoreType.DMA,
        ],
    )


_BJ = 128


def _dot_body(seq_ref, tgt_ref, out_ref):
    s = seq_ref[...].reshape(_BJ, _L, _D)
    t = tgt_ref[...].reshape(_BJ, 1, _D)
    s2 = jnp.sum(s * t, axis=2)
    r = lax.broadcasted_iota(jnp.int32, (_L, _L), 0)
    c = lax.broadcasted_iota(jnp.int32, (_L, _L), 1)
    tri = (c <= r).astype(jnp.float32)
    out_ref[...] = lax.dot_general(
        tri, s2, (((1,), (1,)), ((), ())), preferred_element_type=jnp.float32)


_dot_call = pl.pallas_call(
    _dot_body,
    grid=(_B // _BJ,),
    in_specs=[
        pl.BlockSpec((_BJ * _L, _D), lambda j: (j, 0)),
        pl.BlockSpec((_BJ, _D), lambda j: (j, 0)),
    ],
    out_specs=pl.BlockSpec((_L, _BJ), lambda j: (0, j)),
    out_shape=jax.ShapeDtypeStruct((_L, _B), jnp.float32),
)


_BI = 64


def _bcast_body(dotT_ref, bias_ref, out_ref):
    d = dotT_ref[...]
    b = bias_ref[...]
    for l in range(_L):
        out_ref[l] = d[l:l + 1, :] + b


_bcast_call = pl.pallas_call(
    _bcast_body,
    grid=(_B // _BI,),
    in_specs=[
        pl.BlockSpec((_L, _B), lambda i: (0, 0)),
        pl.BlockSpec((_BI, 1), lambda i: (i, 0)),
    ],
    out_specs=pl.BlockSpec((_L, _BI, _B), lambda i: (0, i, 0)),
    out_shape=jax.ShapeDtypeStruct((_L, _B, _B), jnp.float32),
)


def kernel(item_sequences, item_ids, item_embeddings_weight, item_biases_weight):
    seq = item_sequences.reshape(-1)
    ids = item_ids.reshape(-1)
    seq_rows, tgt_rows = _sc_gather()(item_embeddings_weight, seq, ids)
    dotT = _dot_call(seq_rows, tgt_rows)
    bias_g = item_biases_weight[ids]
    out_phys = _bcast_call(dotT, bias_g)
    return jnp.transpose(out_phys, (1, 2, 0))


# speedup vs baseline: 1.8614x; 1.0126x over previous
import functools

import jax
import jax.numpy as jnp
from jax import lax
from jax.experimental import pallas as pl
from jax.experimental.pallas import tpu as pltpu
from jax.experimental.pallas import tpu_sc as plsc

_B = 1024
_L = 20
_D = 64
_NC = 2
_NS = 16
_NW = _NC * _NS
_BPW = _B // _NW
_SEQ_PW = _BPW * _L
_CH = 128
_NCH = _SEQ_PW // _CH


_CHUNK = 32


def _sc_gather_body(table, seq, ids,
                    seq_rows, tgt_rows,
                    seq_idx_v, ids_v, rows_v, tgt_v, sem):
    wid = lax.axis_index("s") * _NC + lax.axis_index("c")
    jb = wid * _BPW
    sb = wid * _SEQ_PW
    pltpu.sync_copy(seq.at[pl.ds(sb, _SEQ_PW)], seq_idx_v)
    pltpu.sync_copy(ids.at[pl.ds(jb, _BPW)], ids_v)
    pending = []
    for c in range(_SEQ_PW // 16):
        vec = seq_idx_v[pl.ds(c * 16, 16)]
        copies = []
        for u in range(16):
            i = c * 16 + u
            copies.append(pltpu.async_copy(
                table.at[pl.ds(vec[u], 1)], rows_v.at[pl.ds(i, 1)], sem))
        pending.append(copies)
        if len(pending) > 7:
            for cp in pending.pop(0):
                cp.wait()
    pending = [cp for chunk in pending for cp in chunk]
    tcopies = []
    for c in range(_BPW // 16):
        vec = ids_v[pl.ds(c * 16, 16)]
        for u in range(16):
            i = c * 16 + u
            tcopies.append(pltpu.async_copy(
                table.at[pl.ds(vec[u], 1)], tgt_v.at[pl.ds(i, 1)], sem))
    for cp in pending:
        cp.wait()
    for cp in tcopies:
        cp.wait()
    pltpu.sync_copy(rows_v, seq_rows.at[pl.ds(sb, _SEQ_PW)])
    pltpu.sync_copy(tgt_v, tgt_rows.at[pl.ds(jb, _BPW)])


@functools.cache
def _sc_gather():
    return pl.kernel(
        _sc_gather_body,
        out_type=(jax.ShapeDtypeStruct((_B * _L, _D), jnp.float32),
                  jax.ShapeDtypeStruct((_B, _D), jnp.float32)),
        mesh=plsc.VectorSubcoreMesh(core_axis_name="c", subcore_axis_name="s"),
        scratch_types=[
            pltpu.VMEM((_SEQ_PW,), jnp.int32),
            pltpu.VMEM((_BPW,), jnp.int32),
            pltpu.VMEM((_SEQ_PW, _D), jnp.float32),
            pltpu.VMEM((_BPW, _D), jnp.float32),
            pltpu.SemaphoreType.DMA,
        ],
    )


_BJ = 128


def _dot_body(seq_ref, tgt_ref, out_ref):
    s = seq_ref[...].reshape(_BJ, _L, _D)
    t = tgt_ref[...].reshape(_BJ, 1, _D)
    s2 = jnp.sum(s * t, axis=2)
    r = lax.broadcasted_iota(jnp.int32, (_L, _L), 0)
    c = lax.broadcasted_iota(jnp.int32, (_L, _L), 1)
    tri = (c <= r).astype(jnp.float32)
    out_ref[...] = lax.dot_general(
        tri, s2, (((1,), (1,)), ((), ())), preferred_element_type=jnp.float32)


_dot_call = pl.pallas_call(
    _dot_body,
    grid=(_B // _BJ,),
    in_specs=[
        pl.BlockSpec((_BJ * _L, _D), lambda j: (j, 0)),
        pl.BlockSpec((_BJ, _D), lambda j: (j, 0)),
    ],
    out_specs=pl.BlockSpec((_L, _BJ), lambda j: (0, j)),
    out_shape=jax.ShapeDtypeStruct((_L, _B), jnp.float32),
)


_BI = 64


def _bcast_body(dotT_ref, bias_ref, out_ref):
    d = dotT_ref[...]
    b = bias_ref[...]
    for l in range(_L):
        out_ref[l] = d[l:l + 1, :] + b


_bcast_call = pl.pallas_call(
    _bcast_body,
    grid=(_B // _BI,),
    in_specs=[
        pl.BlockSpec((_L, _B), lambda i: (0, 0)),
        pl.BlockSpec((_BI, 1), lambda i: (i, 0)),
    ],
    out_specs=pl.BlockSpec((_L, _BI, _B), lambda i: (0, i, 0)),
    out_shape=jax.ShapeDtypeStruct((_L, _B, _B), jnp.float32),
)


def kernel(item_sequences, item_ids, item_embeddings_weight, item_biases_weight):
    seq = item_sequences.reshape(-1)
    ids = item_ids.reshape(-1)
    seq_rows, tgt_rows = _sc_gather()(item_embeddings_weight, seq, ids)
    dotT = _dot_call(seq_rows, tgt_rows)
    bias_g = item_biases_weight[ids]
    out_phys = _bcast_call(dotT, bias_g)
    return jnp.transpose(out_phys, (1, 2, 0))
